# Initial kernel scaffold; baseline (speedup 1.0000x reference)
import jax
import jax.numpy as jnp
from jax.experimental import pallas as pl


def kernel(x, emb_table, lin_table, bias, W1, b1, W2, b2, W3, b3, W4, b4, W5, b5):
    raise NotImplementedError("write your pallas kernel here")


# speedup vs baseline: 2.0243x; 2.0243x over previous
import functools

import jax
import jax.numpy as jnp
from jax import lax
from jax.experimental import pallas as pl
from jax.experimental.pallas import tpu as pltpu
from jax.experimental.pallas import tpu_sc as plsc

B = 4096
F = 26
D = 32
NW = 32
TOTAL = B * F
ROWS_PER_W = TOTAL // NW
CHUNK = 128
NCHUNK = ROWS_PER_W // CHUNK


def _gather_body(x_hbm, emb_hbm, lin_hbm, emb_out, lin_out,
                 idx_v, rows_v, lin_v, sem_e, sem_l):
    wid = lax.axis_index("s") * 2 + lax.axis_index("c")
    base = wid * ROWS_PER_W
    pltpu.sync_copy(x_hbm.at[wid], idx_v)
    cps_l = []
    for j in range(NCHUNK):
        cps_l.append(pltpu.async_copy(
            lin_hbm.at[idx_v.at[j]], lin_v.at[pl.ds(j * CHUNK, CHUNK)], sem_l))
    half = NCHUNK // 2
    for p in range(2):
        cps_e = []
        for j in range(half):
            cps_e.append(pltpu.async_copy(
                emb_hbm.at[idx_v.at[p * half + j]],
                rows_v.at[pl.ds(j * CHUNK, CHUNK)], sem_e))
        for cp in cps_e:
            cp.wait()
        pltpu.sync_copy(
            rows_v, emb_out.at[pl.ds(base + p * half * CHUNK, half * CHUNK)])
    for cp in cps_l:
        cp.wait()
    pltpu.sync_copy(lin_v, lin_out.at[pl.ds(base, ROWS_PER_W)])


@jax.jit
def _sc_gather(x2d, emb_table, lin_table):
    mesh = plsc.VectorSubcoreMesh(core_axis_name="c", subcore_axis_name="s")
    call = functools.partial(
        pl.kernel,
        out_type=(
            jax.ShapeDtypeStruct((TOTAL, D), jnp.float32),
            jax.ShapeDtypeStruct((TOTAL, 1), jnp.float32),
        ),
        mesh=mesh,
        scratch_types=[
            pltpu.VMEM((NCHUNK, CHUNK), jnp.int32),
            pltpu.VMEM((ROWS_PER_W // 2, D), jnp.float32),
            pltpu.VMEM((ROWS_PER_W, 1), jnp.float32),
            pltpu.SemaphoreType.DMA,
            pltpu.SemaphoreType.DMA,
        ],
        compiler_params=pltpu.CompilerParams(use_tc_tiling_on_sc=False),
    )(_gather_body)
    return call(x2d, emb_table, lin_table)


BB = 512


def _mlp_body(g_ref, ling_ref, bias_ref, w1, b1, w2, b2, w3, b3, w4, b4, w5, b5,
              out_ref):
    h = g_ref[...]
    h = jnp.maximum(jnp.dot(h, w1[...], preferred_element_type=jnp.float32)
                    + b1[...][None, :], 0.0)
    h = jnp.maximum(jnp.dot(h, w2[...], preferred_element_type=jnp.float32)
                    + b2[...][None, :], 0.0)
    h = jnp.maximum(jnp.dot(h, w3[...], preferred_element_type=jnp.float32)
                    + b3[...][None, :], 0.0)
    h = jnp.maximum(jnp.dot(h, w4[...], preferred_element_type=jnp.float32)
                    + b4[...][None, :], 0.0)
    deep = jnp.dot(h, w5[...], preferred_element_type=jnp.float32) + b5[...][None, :]
    wide = jnp.sum(ling_ref[...], axis=1, keepdims=True)
    out_ref[...] = jax.nn.sigmoid(bias_ref[...][None, :] + wide + deep)


@jax.jit
def _tc_mlp(g, ling, bias, W1, b1, W2, b2, W3, b3, W4, b4, W5, b5):
    full = lambda shape: pl.BlockSpec(shape, lambda i: (0,) * len(shape))
    return pl.pallas_call(
        _mlp_body,
        grid=(B // BB,),
        in_specs=[
            pl.BlockSpec((BB, F * D), lambda i: (i, 0)),
            pl.BlockSpec((BB, F), lambda i: (i, 0)),
            full((1,)),
            full(W1.shape), full(b1.shape),
            full(W2.shape), full(b2.shape),
            full(W3.shape), full(b3.shape),
            full(W4.shape), full(b4.shape),
            full(W5.shape), full(b5.shape),
        ],
        out_specs=pl.BlockSpec((BB, 1), lambda i: (i, 0)),
        out_shape=jax.ShapeDtypeStruct((B, 1), jnp.float32),
        compiler_params=pltpu.CompilerParams(
            dimension_semantics=("arbitrary",),
        ),
    )(g, ling, bias, W1, b1, W2, b2, W3, b3, W4, b4, W5, b5)


def kernel(x, emb_table, lin_table, bias, W1, b1, W2, b2, W3, b3, W4, b4, W5, b5):
    x2d = x.reshape(NW, NCHUNK, CHUNK).astype(jnp.int32)
    gathered, lin_g = _sc_gather(x2d, emb_table, lin_table)
    g = gathered.reshape(B, F * D)
    ling = lin_g.reshape(B, F)
    return _tc_mlp(g, ling, bias, W1, b1, W2, b2, W3, b3, W4, b4, W5, b5)


# speedup vs baseline: 5.0057x; 2.4728x over previous
import functools

import jax
import jax.numpy as jnp
from jax import lax
from jax.experimental import pallas as pl
from jax.experimental.pallas import tpu as pltpu
from jax.experimental.pallas import tpu_sc as plsc

B = 4096
F = 26
D = 32
NW = 32
TOTAL = B * F
ROWS_PER_W = TOTAL // NW
B_PER_W = B // NW
CHUNK = 128
NCHUNK = ROWS_PER_W // CHUNK
HALF = NCHUNK // 2
PAIR_COLS = 2 * F * D
EMB_OUT_ROWS = TOTAL * D // PAIR_COLS
ROWS_OUT_PER_PASS = HALF * CHUNK * D // PAIR_COLS


def _gather_body(x_hbm, emb_hbm, lin_hbm, emb_out, wide_out,
                 idx_v, rows_v, lin_v, wide_v, sem_e, sem_l):
    wid = lax.axis_index("s") * 2 + lax.axis_index("c")
    pltpu.sync_copy(x_hbm.at[wid], idx_v)
    cps_l = []
    for j in range(NCHUNK):
        cps_l.append(pltpu.async_copy(
            lin_hbm.at[idx_v.at[j]], lin_v.at[pl.ds(j * CHUNK, CHUNK)], sem_l))
    for p in range(2):
        cps_e = []
        for j in range(HALF):
            cps_e.append(pltpu.async_copy(
                emb_hbm.at[idx_v.at[p * HALF + j]], rows_v.at[j], sem_e))
        for cp in cps_e:
            cp.wait()
        pltpu.sync_copy(rows_v, emb_out.at[pl.ds(wid * NCHUNK + p * HALF, HALF)])
    for cp in cps_l:
        cp.wait()
    lane = lax.broadcasted_iota(jnp.int32, (16,), 0)
    for g in range(B_PER_W // 16):
        row0 = (g * 16 + lane) * F
        acc = jnp.zeros((16,), jnp.float32)
        for j in range(F):
            acc = acc + plsc.load_gather(lin_v, [row0 + j])
        wide_v[pl.ds(g * 16, 16)] = acc
    pltpu.sync_copy(wide_v, wide_out.at[pl.ds(wid * B_PER_W, B_PER_W)])


@jax.jit
def _sc_gather(x3d, emb_table, lin_table):
    mesh = plsc.VectorSubcoreMesh(core_axis_name="c", subcore_axis_name="s")
    call = functools.partial(
        pl.kernel,
        out_type=(
            jax.ShapeDtypeStruct((NW * NCHUNK, CHUNK, D), jnp.float32),
            jax.ShapeDtypeStruct((B,), jnp.float32),
        ),
        mesh=mesh,
        scratch_types=[
            pltpu.VMEM((NCHUNK, CHUNK), jnp.int32),
            pltpu.VMEM((HALF, CHUNK, D), jnp.float32),
            pltpu.VMEM((ROWS_PER_W,), jnp.float32),
            pltpu.VMEM((B_PER_W,), jnp.float32),
            pltpu.SemaphoreType.DMA,
            pltpu.SemaphoreType.DMA,
        ],
        compiler_params=pltpu.CompilerParams(
            use_tc_tiling_on_sc=False, needs_layout_passes=False),
    )(_gather_body)
    return call(x3d, emb_table, lin_table)


BR = 256


def _mlp_body(g_ref, wide_ref, bias_ref, w1, b1, w2, b2, w3, b3, w4, b4, w5, b5,
              out_ref):
    h = g_ref[...]
    h = jnp.maximum(jnp.dot(h, w1[...], preferred_element_type=jnp.float32)
                    + b1[...][None, :], 0.0)
    h = jnp.maximum(jnp.dot(h, w2[...], preferred_element_type=jnp.float32)
                    + b2[...][None, :], 0.0)
    h = jnp.maximum(jnp.dot(h, w3[...], preferred_element_type=jnp.float32)
                    + b3[...][None, :], 0.0)
    h = jnp.maximum(jnp.dot(h, w4[...], preferred_element_type=jnp.float32)
                    + b4[...][None, :], 0.0)
    deep = jnp.dot(h, w5[...], preferred_element_type=jnp.float32) + b5[...][None, :]
    out_ref[...] = jax.nn.sigmoid(bias_ref[...][None, :] + wide_ref[...] + deep)


def _paired(W):
    z = jnp.zeros_like(W)
    return jnp.concatenate(
        [jnp.concatenate([W, z], axis=1), jnp.concatenate([z, W], axis=1)], axis=0)


@jax.jit
def _tc_mlp(g2, wide2, bias, W1, b1, W2, b2, W3, b3, W4, b4, W5, b5):
    W1x, W2x, W3x, W4x, W5x = map(_paired, (W1, W2, W3, W4, W5))
    b1x, b2x, b3x, b4x, b5x = (jnp.concatenate([b, b]) for b in (b1, b2, b3, b4, b5))
    full = lambda shape: pl.BlockSpec(shape, lambda i: (0,) * len(shape))
    out2 = pl.pallas_call(
        _mlp_body,
        grid=(EMB_OUT_ROWS // BR,),
        in_specs=[
            pl.BlockSpec((BR, PAIR_COLS), lambda i: (i, 0)),
            pl.BlockSpec((BR, 2), lambda i: (i, 0)),
            full((1,)),
            full(W1x.shape), full(b1x.shape),
            full(W2x.shape), full(b2x.shape),
            full(W3x.shape), full(b3x.shape),
            full(W4x.shape), full(b4x.shape),
            full(W5x.shape), full(b5x.shape),
        ],
        out_specs=pl.BlockSpec((BR, 2), lambda i: (i, 0)),
        out_shape=jax.ShapeDtypeStruct((EMB_OUT_ROWS, 2), jnp.float32),
        compiler_params=pltpu.CompilerParams(
            dimension_semantics=("arbitrary",),
        ),
    )(g2, wide2, bias, W1x, b1x, W2x, b2x, W3x, b3x, W4x, b4x, W5x, b5x)
    return out2.reshape(B, 1)


def kernel(x, emb_table, lin_table, bias, W1, b1, W2, b2, W3, b3, W4, b4, W5, b5):
    x3d = x.reshape(NW, NCHUNK, CHUNK).astype(jnp.int32)
    g3, wide = _sc_gather(x3d, emb_table, lin_table.reshape(lin_table.shape[0]))
    g2 = g3.reshape(EMB_OUT_ROWS, PAIR_COLS)
    wide2 = wide.reshape(B // 2, 2)
    return _tc_mlp(g2, wide2, bias, W1, b1, W2, b2, W3, b3, W4, b4, W5, b5)
